# baseline (speedup 1.0000x reference)
import functools

import jax
import jax.numpy as jnp
from jax import lax
from jax.experimental import pallas as pl
from jax.experimental.pallas import tpu as pltpu
from jax.experimental.pallas import tpu_sc as plsc

_FPAD = 32
_CH = 1024
_NBUF = 2


def _build(n, d, nc, ns):
    nw = nc * ns
    ipw = n // nw
    nch = ipw // _CH
    mesh = plsc.VectorSubcoreMesh(core_axis_name="c", subcore_axis_name="s")

    @functools.partial(
        pl.kernel,
        out_type=jax.ShapeDtypeStruct((n, d), jnp.float32),
        mesh=mesh,
        scratch_types=[
            pltpu.VMEM((nch, _CH), jnp.int32),
            pltpu.VMEM((_NBUF, _CH, d), jnp.float32),
            pltpu.SemaphoreType.DMA,
            pltpu.SemaphoreType.DMA,
            pltpu.SemaphoreType.DMA,
            pltpu.SemaphoreType.DMA,
        ],
        compiler_params=pltpu.CompilerParams(use_tc_tiling_on_sc=False),
    )
    def run(idx_hbm, table_hbm, out_hbm, idx_v, rows_v, g0, g1, w0, w1):
        wid = lax.axis_index("s") * nc + lax.axis_index("c")
        pltpu.sync_copy(idx_hbm.at[pl.ds(wid * nch, nch)], idx_v)
        base = wid * ipw
        gsem = (g0, g1)
        wsem = (w0, w1)

        def gather(c, b):
            return pltpu.async_copy(
                table_hbm.at[idx_v.at[c]], rows_v.at[b], gsem[b])

        def write(c, b):
            return pltpu.async_copy(
                rows_v.at[b], out_hbm.at[pl.ds(base + c * _CH, _CH)], wsem[b])

        gathers = [None] * nch
        writes = [None] * nch
        for c in range(nch):
            b = c % _NBUF
            if c >= _NBUF:
                writes[c - _NBUF].wait()
            gathers[c] = gather(c, b)
            if c >= 1:
                gathers[c - 1].wait()
                writes[c - 1] = write(c - 1, (c - 1) % _NBUF)
        gathers[nch - 1].wait()
        writes[nch - 1] = write(nch - 1, (nch - 1) % _NBUF)
        writes[nch - 2].wait()
        writes[nch - 1].wait()

    return run


def kernel(indices, weight):
    b, f = indices.shape
    v, d = weight.shape
    idx_pad = jnp.pad(indices.astype(jnp.int32), ((0, 0), (0, _FPAD - f)))
    n = b * _FPAD
    info = plsc.get_sparse_core_info()
    nw = info.num_cores * info.num_subcores
    idx2d = idx_pad.reshape(nw * (n // nw // _CH), _CH)
    run = _build(n, d, info.num_cores, info.num_subcores)
    out = run(idx2d, weight)
    return out.reshape(b, _FPAD, d)[:, :f, :]


# speedup vs baseline: 1.9963x; 1.9963x over previous
import functools

import jax
import jax.numpy as jnp
from jax import lax
from jax.experimental import pallas as pl
from jax.experimental.pallas import tpu as pltpu
from jax.experimental.pallas import tpu_sc as plsc

_BW = 512


def _build(nb, nf, d, nc, ns):
    mesh = plsc.VectorSubcoreMesh(core_axis_name="c", subcore_axis_name="s")
    assert nf % 2 == 0

    @functools.partial(
        pl.kernel,
        out_type=jax.ShapeDtypeStruct((nf, d, nb), jnp.float32),
        mesh=mesh,
        scratch_types=[
            pltpu.VMEM((nf, _BW), jnp.int32),
            pltpu.VMEM((_BW, d), jnp.float32),
            pltpu.VMEM((_BW, d), jnp.float32),
            pltpu.VMEM((d, _BW), jnp.float32),
            pltpu.VMEM((d, _BW), jnp.float32),
            pltpu.SemaphoreType.DMA,
            pltpu.SemaphoreType.DMA,
            pltpu.SemaphoreType.DMA,
            pltpu.SemaphoreType.DMA,
        ],
        compiler_params=pltpu.CompilerParams(
            use_tc_tiling_on_sc=False, needs_layout_passes=False),
    )
    def run(idx_hbm, table_hbm, out_hbm, idx_v, a0, a1, b0, b1, g0, g1, w0, w1):
        wid = lax.axis_index("s") * nc + lax.axis_index("c")
        base = wid * _BW
        pltpu.sync_copy(idx_hbm.at[:, pl.ds(base, _BW)], idx_v)
        iota = lax.iota(jnp.int32, 16)

        def transpose(a, b):
            def tbody(q, carry):
                row = q * 16 + iota
                for dd in range(d):
                    col = jnp.full((16,), dd, jnp.int32)
                    v = plsc.load_gather(a, [row, col])
                    b[dd, pl.ds(q * 16, 16)] = v
                return carry

            lax.fori_loop(0, _BW // 16, tbody, 0)

        def step(f, a, bb, gsem, wsem):
            ga = pltpu.async_copy(table_hbm.at[idx_v.at[f]], a, gsem)
            return ga

        def body(i, carry):
            f0 = 2 * i
            f1 = f0 + 1
            ga0 = pltpu.async_copy(table_hbm.at[idx_v.at[f0]], a0, g0)
            ga1 = pltpu.async_copy(table_hbm.at[idx_v.at[f1]], a1, g1)
            ga0.wait()
            transpose(a0, b0)
            wb0 = pltpu.async_copy(b0, out_hbm.at[f0, :, pl.ds(base, _BW)], w0)
            ga1.wait()
            transpose(a1, b1)
            wb1 = pltpu.async_copy(b1, out_hbm.at[f1, :, pl.ds(base, _BW)], w1)
            wb0.wait()
            wb1.wait()
            return carry

        lax.fori_loop(0, nf // 2, body, 0)

    return run


def kernel(indices, weight):
    nb, nf = indices.shape
    v, d = weight.shape
    info = plsc.get_sparse_core_info()
    idx_t = indices.T.astype(jnp.int32)
    run = _build(nb, nf, d, info.num_cores, info.num_subcores)
    out_t = run(idx_t, weight)
    return out_t.transpose(2, 0, 1)
